# Initial kernel scaffold; baseline (speedup 1.0000x reference)
import jax
import jax.numpy as jnp
from jax.experimental import pallas as pl


def kernel(x, table):
    raise NotImplementedError("write your pallas kernel here")


# speedup vs baseline: 2.4146x; 2.4146x over previous
import functools
import math

import jax
import jax.numpy as jnp
from jax import lax
from jax.experimental import pallas as pl
from jax.experimental.pallas import tpu as pltpu
from jax.experimental.pallas import tpu_sc as plsc

_VOCAB = 100000
_D = 128
_B = 4096
_L = 50
_NTOK = _B * _L
_NC = 2
_NS = 16
_NW = _NC * _NS
_CHUNK = 128
_ROWS_PER_W = _NTOK // _NW
_CHUNKS_PER_W = _ROWS_PER_W // _CHUNK
_IDX_ROWS = _NTOK // _CHUNK
_SCALE = math.sqrt(float(_D))


def _emb_kernel(idx_hbm, table_hbm, out_hbm, idx_v, rows_v, gsem):
    wid = lax.axis_index("s") * _NC + lax.axis_index("c")
    tok0 = wid * _ROWS_PER_W

    pltpu.sync_copy(idx_hbm.at[pl.ds(tok0, _ROWS_PER_W)], idx_v)

    def body(j, carry):
        off = pl.multiple_of(j * _CHUNK, _CHUNK)
        pltpu.async_copy(
            table_hbm.at[idx_v.at[pl.ds(off, _CHUNK)]], rows_v, gsem
        ).wait()

        def scale_row(r, c2):
            for c in range(_D // 16):
                rows_v[r, pl.ds(c * 16, 16)] = (
                    rows_v[r, pl.ds(c * 16, 16)] * _SCALE
                )
            return c2

        lax.fori_loop(0, _CHUNK, scale_row, 0, unroll=False)

        out_row0 = pl.multiple_of(tok0 + j * _CHUNK, _CHUNK)
        pltpu.sync_copy(rows_v, out_hbm.at[pl.ds(out_row0, _CHUNK)])
        return carry

    lax.fori_loop(0, _CHUNKS_PER_W, body, 0, unroll=False)


@jax.jit
def _run(x2, table):
    mesh = plsc.VectorSubcoreMesh(core_axis_name="c", subcore_axis_name="s")
    f = functools.partial(
        pl.kernel,
        out_type=jax.ShapeDtypeStruct((_NTOK, _D), jnp.float32),
        mesh=mesh,
        scratch_types=[
            pltpu.VMEM((_ROWS_PER_W,), jnp.int32),
            pltpu.VMEM((_CHUNK, _D), jnp.float32),
            pltpu.SemaphoreType.DMA,
        ],
    )(_emb_kernel)
    return f(x2, table)


def kernel(x, table):
    x2 = x.reshape(_NTOK)
    out = _run(x2, table)
    return out.reshape(_B, _L, _D)


# speedup vs baseline: 2.9540x; 1.2234x over previous
import functools
import math

import jax
import jax.numpy as jnp
from jax import lax
from jax.experimental import pallas as pl
from jax.experimental.pallas import tpu as pltpu
from jax.experimental.pallas import tpu_sc as plsc

_VOCAB = 100000
_D = 128
_B = 4096
_L = 50
_NTOK = _B * _L
_NC = 2
_NS = 16
_NW = _NC * _NS
_CHUNK = 128
_ROWS_PER_W = _NTOK // _NW
_NCH = _ROWS_PER_W // _CHUNK
_NBUF = 5
_SCALE = math.sqrt(float(_D))


def _emb_kernel(idx_hbm, table_hbm, out_hbm, idx_v, *scratch):
    rows = scratch[:_NBUF]
    gsem = scratch[_NBUF:2 * _NBUF]
    ssem = scratch[2 * _NBUF:3 * _NBUF]

    wid = lax.axis_index("s") * _NC + lax.axis_index("c")
    tok0 = wid * _ROWS_PER_W

    pltpu.sync_copy(idx_hbm.at[pl.ds(tok0, _ROWS_PER_W)], idx_v)

    def start_gather(j, b):
        off = pl.multiple_of(j * _CHUNK, _CHUNK)
        pltpu.async_copy(
            table_hbm.at[idx_v.at[pl.ds(off, _CHUNK)]], rows[b], gsem[b]
        )

    def out_slice(j):
        row0 = pl.multiple_of(tok0 + j * _CHUNK, _CHUNK)
        return out_hbm.at[pl.ds(row0, _CHUNK)]

    for b in range(_NBUF):
        start_gather(b, b)

    def outer(g, carry):
        for b in range(_NBUF):
            j = g * _NBUF + b
            pltpu.make_async_copy(
                table_hbm.at[idx_v.at[pl.ds(0, _CHUNK)]], rows[b], gsem[b]
            ).wait()

            def scale_row(r, c2, _b=b):
                for c in range(_D // 16):
                    rows[_b][r, pl.ds(c * 16, 16)] = (
                        rows[_b][r, pl.ds(c * 16, 16)] * _SCALE
                    )
                return c2

            lax.fori_loop(0, _CHUNK, scale_row, 0, unroll=False)

            pltpu.async_copy(rows[b], out_slice(j), ssem[b])

            @pl.when(g < (_NCH // _NBUF) - 1)
            def _(b=b, j=j):
                pltpu.make_async_copy(rows[b], out_slice(j), ssem[b]).wait()
                start_gather(j + _NBUF, b)
        return carry

    lax.fori_loop(0, _NCH // _NBUF, outer, 0, unroll=False)

    for b in range(_NBUF):
        pltpu.make_async_copy(rows[b], out_slice(0), ssem[b]).wait()


@jax.jit
def _run(x2, table):
    mesh = plsc.VectorSubcoreMesh(core_axis_name="c", subcore_axis_name="s")
    f = functools.partial(
        pl.kernel,
        out_type=jax.ShapeDtypeStruct((_NTOK, _D), jnp.float32),
        mesh=mesh,
        scratch_types=[pltpu.VMEM((_ROWS_PER_W,), jnp.int32)]
        + [pltpu.VMEM((_CHUNK, _D), jnp.float32) for _ in range(_NBUF)]
        + [pltpu.SemaphoreType.DMA for _ in range(2 * _NBUF)],
    )(_emb_kernel)
    return f(x2, table)


def kernel(x, table):
    x2 = x.reshape(_NTOK)
    out = _run(x2, table)
    return out.reshape(_B, _L, _D)
